# baseline (speedup 1.0000x reference)
import functools

import jax
import jax.numpy as jnp
from jax import lax
from jax.experimental import pallas as pl
from jax.experimental.pallas import tpu as pltpu
from jax.experimental.pallas import tpu_sc as plsc

NC = 2
NS = 16
CH = 128
SPLIT0 = 1.0 if NC == 1 else 0.5


def _sc_mesh():
    return plsc.VectorSubcoreMesh(core_axis_name="c", subcore_axis_name="s",
                                  num_cores=NC)


def _sc_degree(dst2, n_pad, q0, q1, rows_pt):
    q_max = max(q0, q1)

    @functools.partial(
        pl.kernel,
        out_type=jax.ShapeDtypeStruct((NC * n_pad,), jnp.float32),
        mesh=_sc_mesh(),
        compiler_params=pltpu.CompilerParams(use_tc_tiling_on_sc=False),
        scratch_types=[
            pltpu.VMEM((q_max, CH), jnp.int32),
            pltpu.VMEM((CH,), jnp.float32),
            pltpu.VMEM((rows_pt,), jnp.float32),
            pltpu.VMEM_SHARED((n_pad,), jnp.float32),
            pltpu.SemaphoreType.DMA,
        ],
    )
    def k(dst_hbm, out_hbm, dst_v, ones_v, stage_v, deg_s, dsem):
        ci = lax.axis_index("c")
        si = lax.axis_index("s")
        for i in range(CH // 16):
            ones_v[pl.ds(i * 16, 16)] = jnp.ones((16,), jnp.float32)

        def zrow(r, _):
            stage_v[pl.ds(r * 16, 16)] = jnp.zeros((16,), jnp.float32)
            return ()

        lax.fori_loop(0, rows_pt // 16, zrow, ())
        pltpu.sync_copy(stage_v, deg_s.at[pl.ds(si * rows_pt, rows_pt)])

        def run(q, start):
            pltpu.sync_copy(dst_hbm.at[pl.ds(start, q)],
                            dst_v.at[pl.ds(0, q)])
            plsc.subcore_barrier()
            la = 8

            def start_j(j):
                pltpu.async_copy(ones_v, deg_s.at[dst_v.at[j]], dsem,
                                 add=True)

            def drain_j(j):
                pltpu.make_async_copy(ones_v, deg_s.at[dst_v.at[j]],
                                      dsem).wait()

            for j in range(la):
                start_j(j)

            def body(j, _):
                drain_j(j)
                start_j(j + la)
                return ()

            lax.fori_loop(0, q - la, body, ())

            def tail(j, _):
                drain_j(j)
                return ()

            lax.fori_loop(q - la, q, tail, ())

        if NC == 1:
            run(q0, si * q0)
        else:
            @pl.when(ci == 0)
            def _():
                run(q0, si * q0)

            @pl.when(ci == 1)
            def _():
                run(q1, NS * q0 + si * q1)

        plsc.subcore_barrier()
        pltpu.sync_copy(deg_s.at[pl.ds(si * rows_pt, rows_pt)], stage_v)
        pltpu.sync_copy(stage_v,
                        out_hbm.at[pl.ds(ci * n_pad + si * rows_pt, rows_pt)])

    return k(dst2).reshape(NC, n_pad)


def _sc_aggregate_packed(src2, dst2, hp, n_pad, q0, q1, rows_pt):
    WP = hp.shape[1]
    W = 2 * WP
    q_max = max(q0, q1)

    @functools.partial(
        pl.kernel,
        out_type=jax.ShapeDtypeStruct((NC, n_pad, W), jnp.float32),
        mesh=_sc_mesh(),
        compiler_params=pltpu.CompilerParams(use_tc_tiling_on_sc=False),
        scratch_types=[
            pltpu.VMEM((q_max, CH), jnp.int32),
            pltpu.VMEM((q_max, CH), jnp.int32),
            [pltpu.VMEM((CH, WP), jnp.int32) for _ in range(4)],
            [pltpu.VMEM((CH, W), jnp.float32) for _ in range(2)],
            pltpu.VMEM_SHARED((n_pad, W), jnp.float32),
            [pltpu.SemaphoreType.DMA for _ in range(4)],
            [pltpu.SemaphoreType.DMA for _ in range(2)],
        ],
    )
    def k(src_hbm, dst_hbm, hp_hbm, out_hbm,
          src_v, dst_v, pbufs, fbufs, acc_s, gsems, ssems):
        ci = lax.axis_index("c")
        si = lax.axis_index("s")
        n_pieces = -(-rows_pt // CH)
        pieces = [(p * CH, min(CH, rows_pt - p * CH)) for p in range(n_pieces)]

        def zrow(r, _):
            for b in range(2):
                for c in range(W // 16):
                    fbufs[b][r, pl.ds(c * 16, 16)] = jnp.zeros(
                        (16,), jnp.float32)
            return ()

        lax.fori_loop(0, CH, zrow, ())
        init_copies = []
        for p, (off, sz) in enumerate(pieces):
            sem = ssems[p % 2] if p < 2 else gsems[p - 2]
            cp = pltpu.async_copy(fbufs[p % 2].at[pl.ds(0, sz)],
                                  acc_s.at[pl.ds(si * rows_pt + off, sz)],
                                  sem)
            init_copies.append(cp)

        def gather(j, b):
            pltpu.async_copy(hp_hbm.at[src_v.at[j]], pbufs[b], gsems[b])

        def gather_wait(j, b):
            pltpu.make_async_copy(
                hp_hbm.at[src_v.at[j]], pbufs[b], gsems[b]).wait()

        def scatter(j, f):
            pltpu.async_copy(fbufs[f], acc_s.at[dst_v.at[j]], ssems[f],
                             add=True)

        def scatter_wait(j, f):
            pltpu.make_async_copy(
                fbufs[f], acc_s.at[dst_v.at[j]], ssems[f]).wait()

        mask = jnp.full((16,), -65536, jnp.int32)
        sixteen = jnp.full((16,), 16, jnp.int32)

        def convert(b, f):
            def crow(r, _):
                for c in range(WP // 16):
                    w = pbufs[b][r, pl.ds(c * 16, 16)]
                    lo = lax.bitcast_convert_type(lax.shift_left(w, sixteen), jnp.float32)
                    hi = lax.bitcast_convert_type(lax.bitwise_and(w, mask), jnp.float32)
                    fbufs[f][r, pl.ds(c * 16, 16)] = lo
                    fbufs[f][r, pl.ds(WP + c * 16, 16)] = hi
                return ()

            lax.fori_loop(0, CH, crow, ())

        def run(q, start):
            pltpu.sync_copy(src_hbm.at[pl.ds(start, q)],
                            src_v.at[pl.ds(0, q)])
            pltpu.sync_copy(dst_hbm.at[pl.ds(start, q)],
                            dst_v.at[pl.ds(0, q)])
            for cp in init_copies:
                cp.wait()
            plsc.subcore_barrier()

            for b in range(4):
                gather(b, b)

            def body(i, _):
                for b in range(4):
                    j = 4 * i + b
                    f = b % 2
                    gather_wait(j, b)

                    @pl.when(j >= 2)
                    def _():
                        scatter_wait(j - 2, f)

                    convert(b, f)
                    scatter(j, f)

                    @pl.when(j + 4 < q)
                    def _():
                        gather(j + 4, b)

                return ()

            lax.fori_loop(0, q // 4, body, ())
            scatter_wait(q - 2, 0)
            scatter_wait(q - 1, 1)

        if NC == 1:
            run(q0, si * q0)
        else:
            @pl.when(ci == 0)
            def _():
                run(q0, si * q0)

            @pl.when(ci == 1)
            def _():
                run(q1, NS * q0 + si * q1)

        plsc.subcore_barrier()

        def drain_sv(p):
            off, sz = pieces[p]
            return pltpu.async_copy(
                acc_s.at[pl.ds(si * rows_pt + off, sz)],
                fbufs[p % 2].at[pl.ds(0, sz)], gsems[p % 2])

        def drain_vh(p):
            off, sz = pieces[p]
            return pltpu.async_copy(
                fbufs[p % 2].at[pl.ds(0, sz)],
                out_hbm.at[ci, pl.ds(si * rows_pt + off, sz)], ssems[p % 2])

        P = len(pieces)
        sv = {p: drain_sv(p) for p in range(min(2, P))}
        vh = {}
        for p in range(min(2, P)):
            sv[p].wait()
            vh[p] = drain_vh(p)
        for p in range(2, P):
            vh[p - 2].wait()
            sv[p] = drain_sv(p)
            sv[p].wait()
            vh[p] = drain_vh(p)
        for p in range(max(0, P - 2), P):
            vh[p].wait()

    return k(src2, dst2, hp)


def _sc_aggregate(src2, dst2, h, n_pad, q0, q1, rows_pt):
    W = h.shape[1]
    q_max = max(q0, q1)

    @functools.partial(
        pl.kernel,
        out_type=jax.ShapeDtypeStruct((NC, n_pad, W), jnp.float32),
        mesh=_sc_mesh(),
        compiler_params=pltpu.CompilerParams(use_tc_tiling_on_sc=False),
        scratch_types=[
            pltpu.VMEM((q_max, CH), jnp.int32),
            pltpu.VMEM((q_max, CH), jnp.int32),
            [pltpu.VMEM((CH, W), jnp.float32) for _ in range(4)],
            pltpu.VMEM_SHARED((n_pad, W), jnp.float32),
            [pltpu.SemaphoreType.DMA for _ in range(4)],
            [pltpu.SemaphoreType.DMA for _ in range(4)],
        ],
    )
    def k(src_hbm, dst_hbm, h_hbm, out_hbm,
          src_v, dst_v, bufs, acc_s, gsems, ssems):
        ci = lax.axis_index("c")
        si = lax.axis_index("s")
        n_pieces = -(-rows_pt // CH)
        pieces = [(p * CH, min(CH, rows_pt - p * CH)) for p in range(n_pieces)]

        def zrow(r, _):
            for b in range(4):
                for c in range(W // 16):
                    bufs[b][r, pl.ds(c * 16, 16)] = jnp.zeros(
                        (16,), jnp.float32)
            return ()

        lax.fori_loop(0, CH, zrow, ())
        init_copies = []
        for p, (off, sz) in enumerate(pieces):
            sem = gsems[p % 4] if p < 4 else ssems[p - 4]
            cp = pltpu.async_copy(bufs[p % 4].at[pl.ds(0, sz)],
                                  acc_s.at[pl.ds(si * rows_pt + off, sz)],
                                  sem)
            init_copies.append(cp)

        def gather(j, b):
            pltpu.async_copy(h_hbm.at[src_v.at[j]], bufs[b], gsems[b])

        def gather_wait(j, b):
            pltpu.make_async_copy(
                h_hbm.at[src_v.at[j]], bufs[b], gsems[b]).wait()

        def scatter(j, b):
            pltpu.async_copy(bufs[b], acc_s.at[dst_v.at[j]], ssems[b],
                             add=True)

        def scatter_wait(j, b):
            pltpu.make_async_copy(
                bufs[b], acc_s.at[dst_v.at[j]], ssems[b]).wait()

        def run(q, start):
            pltpu.sync_copy(src_hbm.at[pl.ds(start, q)],
                            src_v.at[pl.ds(0, q)])
            pltpu.sync_copy(dst_hbm.at[pl.ds(start, q)],
                            dst_v.at[pl.ds(0, q)])
            for cp in init_copies:
                cp.wait()
            plsc.subcore_barrier()

            for b in range(4):
                gather(b, b)

            def body(i, _):
                for b in range(4):
                    j = 4 * i + b
                    gather_wait(j, b)
                    scatter(j, b)

                    @pl.when(j + 4 < q)
                    def _():
                        scatter_wait(j, b)
                        gather(j + 4, b)

                return ()

            lax.fori_loop(0, q // 4, body, ())
            for b in range(4):
                scatter_wait(q - 4 + b, b)

        if NC == 1:
            run(q0, si * q0)
        else:
            @pl.when(ci == 0)
            def _():
                run(q0, si * q0)

            @pl.when(ci == 1)
            def _():
                run(q1, NS * q0 + si * q1)

        plsc.subcore_barrier()
        def drain_sv(p):
            off, sz = pieces[p]
            return pltpu.async_copy(
                acc_s.at[pl.ds(si * rows_pt + off, sz)],
                bufs[p % 4].at[pl.ds(0, sz)], gsems[p % 4])

        def drain_vh(p):
            off, sz = pieces[p]
            return pltpu.async_copy(
                bufs[p % 4].at[pl.ds(0, sz)],
                out_hbm.at[ci, pl.ds(si * rows_pt + off, sz)], ssems[p % 4])

        P = len(pieces)
        sv = {p: drain_sv(p) for p in range(min(4, P))}
        vh = {}
        for p in range(min(4, P)):
            sv[p].wait()
            vh[p] = drain_vh(p)
        for p in range(4, P):
            vh[p - 4].wait()
            sv[p] = drain_sv(p)
            sv[p].wait()
            vh[p] = drain_vh(p)
        for p in range(max(0, P - 4), P):
            vh[p].wait()

    return k(src2, dst2, h)


def _tc_matmul1(x, W1, rb):
    n, d_in = x.shape
    d_hid = W1.shape[1]

    def body(x_ref, w_ref, h_ref):
        h_ref[...] = jnp.dot(x_ref[...], w_ref[...],
                             preferred_element_type=jnp.float32)

    return pl.pallas_call(
        body,
        grid=(n // rb,),
        in_specs=[
            pl.BlockSpec((rb, d_in), lambda i: (i, 0)),
            pl.BlockSpec((d_in, d_hid), lambda i: (0, 0)),
        ],
        out_specs=pl.BlockSpec((rb, d_hid), lambda i: (i, 0)),
        out_shape=jax.ShapeDtypeStruct((n, d_hid), jnp.float32),
    )(x, W1)


def _tc_scale1(degp, h1, rb):
    n, d_hid = h1.shape
    hw = d_hid // 2

    def body(degp_ref, h_ref, hsp_ref, dis_ref):
        deg = sum(degp_ref[c] for c in range(NC)) + 1.0
        dis = lax.rsqrt(jnp.maximum(deg, 1.0))
        hs = h_ref[...] * dis
        bits = lax.bitcast_convert_type(
            hs.astype(jnp.bfloat16), jnp.uint16).astype(jnp.uint32)
        word = jnp.bitwise_or(bits[:, :hw],
                              jnp.left_shift(bits[:, hw:], 16))
        hsp_ref[...] = lax.bitcast_convert_type(word, jnp.int32)
        dis_ref[...] = dis

    return pl.pallas_call(
        body,
        grid=(n // rb,),
        in_specs=[
            pl.BlockSpec((NC, rb, 1), lambda i: (0, i, 0)),
            pl.BlockSpec((rb, d_hid), lambda i: (i, 0)),
        ],
        out_specs=[
            pl.BlockSpec((rb, hw), lambda i: (i, 0)),
            pl.BlockSpec((rb, 1), lambda i: (i, 0)),
        ],
        out_shape=[
            jax.ShapeDtypeStruct((n, hw), jnp.int32),
            jax.ShapeDtypeStruct((n, 1), jnp.float32),
        ],
    )(degp, h1)


def _tc_layer2(aggp, hsp, dis, W2p, b1, rb, wp):
    n, hw = hsp.shape
    d_hid = 2 * hw

    def body(aggp_ref, hsp_ref, dis_ref, w_ref, b_ref, hs2_ref):
        w = hsp_ref[...]
        lo = lax.bitcast_convert_type(jnp.left_shift(w, 16), jnp.float32)
        hi = lax.bitcast_convert_type(
            jnp.bitwise_and(w, jnp.int32(-65536)), jnp.float32)
        hs = jnp.concatenate([lo, hi], axis=1)
        agg = sum(aggp_ref[c] for c in range(NC)) + hs
        r = jnp.maximum(agg * dis_ref[...] + b_ref[...], 0.0)
        h2 = jnp.dot(r, w_ref[...], preferred_element_type=jnp.float32)
        hs2_ref[...] = h2 * dis_ref[...]

    return pl.pallas_call(
        body,
        grid=(n // rb,),
        in_specs=[
            pl.BlockSpec((NC, rb, d_hid), lambda i: (0, i, 0)),
            pl.BlockSpec((rb, hw), lambda i: (i, 0)),
            pl.BlockSpec((rb, 1), lambda i: (i, 0)),
            pl.BlockSpec((d_hid, wp), lambda i: (0, 0)),
            pl.BlockSpec((1, d_hid), lambda i: (0, 0)),
        ],
        out_specs=pl.BlockSpec((rb, wp), lambda i: (i, 0)),
        out_shape=jax.ShapeDtypeStruct((n, wp), jnp.float32),
    )(aggp, hsp, dis, W2p, b1)


def _tc_final(agg2p, hs2w, dis, b2, rb, d_out):
    n, wp = hs2w.shape

    def body(ap_ref, hs2_ref, dis_ref, b_ref, out_ref):
        a = sum(ap_ref[c] for c in range(NC)) + hs2_ref[...]
        o = (a * dis_ref[...])[:, :d_out] + b_ref[...]
        m = jnp.max(o, axis=1, keepdims=True)
        lse = m + jnp.log(jnp.sum(jnp.exp(o - m), axis=1, keepdims=True))
        out_ref[...] = o - lse

    return pl.pallas_call(
        body,
        grid=(n // rb,),
        in_specs=[
            pl.BlockSpec((NC, rb, wp), lambda i: (0, i, 0)),
            pl.BlockSpec((rb, wp), lambda i: (i, 0)),
            pl.BlockSpec((rb, 1), lambda i: (i, 0)),
            pl.BlockSpec((1, d_out), lambda i: (0, 0)),
        ],
        out_specs=pl.BlockSpec((rb, d_out), lambda i: (i, 0)),
        out_shape=jax.ShapeDtypeStruct((n, d_out), jnp.float32),
    )(agg2p, hs2w, dis, b2)


def kernel(x, edge_index, W1, b1, W2, b2):
    n = x.shape[0]
    e = edge_index.shape[1]
    d_hid = W1.shape[1]
    d_out = W2.shape[1]
    wp = 16

    s_tot = -(-(-(-e // (NS * CH))) // 8) * 8
    if NC == 1:
        q0, q1 = s_tot, 0
    else:
        q0 = max(4, int(round(s_tot * SPLIT0 / 4)) * 4)
        q1 = s_tot - q0
    e_pad = NS * s_tot * CH
    n_pad = -(-(n + 1) // (NS * 16)) * (NS * 16)
    rows_pt = n_pad // NS
    rb = 1000 if n % 1000 == 0 else 8

    src = edge_index[0]
    dst = edge_index[1]
    src2 = jnp.concatenate(
        [src, jnp.zeros((e_pad - e,), jnp.int32)]).reshape(NS * s_tot, CH)
    dst2 = jnp.concatenate(
        [dst, jnp.full((e_pad - e,), n, jnp.int32)]).reshape(NS * s_tot, CH)

    W2p = jnp.pad(W2, ((0, 0), (0, wp - d_out)))

    h1 = _tc_matmul1(x, W1, rb)
    degp = _sc_degree(dst2, n_pad, q0, q1, rows_pt)
    degp3 = degp[:, :n, None]
    hsp, dis = _tc_scale1(degp3, h1, rb)
    aggp = _sc_aggregate_packed(src2, dst2, hsp, n_pad, q0, q1, rows_pt)
    hs2w = _tc_layer2(aggp, hsp, dis, W2p, b1.reshape(1, d_hid), rb, wp)
    agg2p = _sc_aggregate(src2, dst2, hs2w, n_pad, q0, q1, rows_pt)
    return _tc_final(agg2p, hs2w, dis, b2.reshape(1, d_out), rb, d_out)


# speedup vs baseline: 1.0313x; 1.0313x over previous
import functools

import jax
import jax.numpy as jnp
from jax import lax
from jax.experimental import pallas as pl
from jax.experimental.pallas import tpu as pltpu
from jax.experimental.pallas import tpu_sc as plsc

NC = 2
NS = 16
CH = 128
SPLIT0 = 1.0 if NC == 1 else 0.55


def _sc_mesh():
    return plsc.VectorSubcoreMesh(core_axis_name="c", subcore_axis_name="s",
                                  num_cores=NC)


def _sc_degree(dst2, n_pad, q0, q1, rows_pt):
    q_max = max(q0, q1)

    @functools.partial(
        pl.kernel,
        out_type=jax.ShapeDtypeStruct((NC * n_pad,), jnp.float32),
        mesh=_sc_mesh(),
        compiler_params=pltpu.CompilerParams(use_tc_tiling_on_sc=False),
        scratch_types=[
            pltpu.VMEM((q_max, CH), jnp.int32),
            pltpu.VMEM((CH,), jnp.float32),
            pltpu.VMEM((rows_pt,), jnp.float32),
            pltpu.VMEM_SHARED((n_pad,), jnp.float32),
            pltpu.SemaphoreType.DMA,
        ],
    )
    def k(dst_hbm, out_hbm, dst_v, ones_v, stage_v, deg_s, dsem):
        ci = lax.axis_index("c")
        si = lax.axis_index("s")
        for i in range(CH // 16):
            ones_v[pl.ds(i * 16, 16)] = jnp.ones((16,), jnp.float32)

        def zrow(r, _):
            stage_v[pl.ds(r * 16, 16)] = jnp.zeros((16,), jnp.float32)
            return ()

        lax.fori_loop(0, rows_pt // 16, zrow, ())
        pltpu.sync_copy(stage_v, deg_s.at[pl.ds(si * rows_pt, rows_pt)])

        def run(q, start):
            pltpu.sync_copy(dst_hbm.at[pl.ds(start, q)],
                            dst_v.at[pl.ds(0, q)])
            plsc.subcore_barrier()
            la = 8

            def start_j(j):
                pltpu.async_copy(ones_v, deg_s.at[dst_v.at[j]], dsem,
                                 add=True)

            def drain_j(j):
                pltpu.make_async_copy(ones_v, deg_s.at[dst_v.at[j]],
                                      dsem).wait()

            for j in range(la):
                start_j(j)

            def body(j, _):
                drain_j(j)
                start_j(j + la)
                return ()

            lax.fori_loop(0, q - la, body, ())

            def tail(j, _):
                drain_j(j)
                return ()

            lax.fori_loop(q - la, q, tail, ())

        if NC == 1:
            run(q0, si * q0)
        else:
            @pl.when(ci == 0)
            def _():
                run(q0, si * q0)

            @pl.when(ci == 1)
            def _():
                run(q1, NS * q0 + si * q1)

        plsc.subcore_barrier()
        pltpu.sync_copy(deg_s.at[pl.ds(si * rows_pt, rows_pt)], stage_v)
        pltpu.sync_copy(stage_v,
                        out_hbm.at[pl.ds(ci * n_pad + si * rows_pt, rows_pt)])

    return k(dst2).reshape(NC, n_pad)


def _sc_aggregate_packed(src2, dst2, hp, n_pad, q0, q1, rows_pt):
    WP = hp.shape[1]
    W = 2 * WP
    q_max = max(q0, q1)

    @functools.partial(
        pl.kernel,
        out_type=jax.ShapeDtypeStruct((NC, n_pad, W), jnp.float32),
        mesh=_sc_mesh(),
        compiler_params=pltpu.CompilerParams(use_tc_tiling_on_sc=False),
        scratch_types=[
            pltpu.VMEM((q_max, CH), jnp.int32),
            pltpu.VMEM((q_max, CH), jnp.int32),
            [pltpu.VMEM((CH, WP), jnp.int32) for _ in range(4)],
            [pltpu.VMEM((CH, W), jnp.float32) for _ in range(2)],
            pltpu.VMEM_SHARED((n_pad, W), jnp.float32),
            [pltpu.SemaphoreType.DMA for _ in range(4)],
            [pltpu.SemaphoreType.DMA for _ in range(2)],
        ],
    )
    def k(src_hbm, dst_hbm, hp_hbm, out_hbm,
          src_v, dst_v, pbufs, fbufs, acc_s, gsems, ssems):
        ci = lax.axis_index("c")
        si = lax.axis_index("s")
        n_pieces = -(-rows_pt // CH)
        pieces = [(p * CH, min(CH, rows_pt - p * CH)) for p in range(n_pieces)]

        def zrow(r, _):
            for b in range(2):
                for c in range(W // 16):
                    fbufs[b][r, pl.ds(c * 16, 16)] = jnp.zeros(
                        (16,), jnp.float32)
            return ()

        lax.fori_loop(0, CH, zrow, ())
        init_copies = []
        for p, (off, sz) in enumerate(pieces):
            sem = ssems[p % 2] if p < 2 else gsems[p - 2]
            cp = pltpu.async_copy(fbufs[p % 2].at[pl.ds(0, sz)],
                                  acc_s.at[pl.ds(si * rows_pt + off, sz)],
                                  sem)
            init_copies.append(cp)

        def gather(j, b):
            pltpu.async_copy(hp_hbm.at[src_v.at[j]], pbufs[b], gsems[b])

        def gather_wait(j, b):
            pltpu.make_async_copy(
                hp_hbm.at[src_v.at[j]], pbufs[b], gsems[b]).wait()

        def scatter(j, f):
            pltpu.async_copy(fbufs[f], acc_s.at[dst_v.at[j]], ssems[f],
                             add=True)

        def scatter_wait(j, f):
            pltpu.make_async_copy(
                fbufs[f], acc_s.at[dst_v.at[j]], ssems[f]).wait()

        mask = jnp.full((16,), -65536, jnp.int32)
        sixteen = jnp.full((16,), 16, jnp.int32)

        def convert(b, f):
            def crow(r, _):
                for c in range(WP // 16):
                    w = pbufs[b][r, pl.ds(c * 16, 16)]
                    lo = lax.bitcast_convert_type(lax.shift_left(w, sixteen), jnp.float32)
                    hi = lax.bitcast_convert_type(lax.bitwise_and(w, mask), jnp.float32)
                    fbufs[f][r, pl.ds(c * 16, 16)] = lo
                    fbufs[f][r, pl.ds(WP + c * 16, 16)] = hi
                return ()

            lax.fori_loop(0, CH, crow, ())

        def run(q, start):
            pltpu.sync_copy(src_hbm.at[pl.ds(start, q)],
                            src_v.at[pl.ds(0, q)])
            pltpu.sync_copy(dst_hbm.at[pl.ds(start, q)],
                            dst_v.at[pl.ds(0, q)])
            for cp in init_copies:
                cp.wait()
            plsc.subcore_barrier()

            for b in range(4):
                gather(b, b)

            def body(i, _):
                for b in range(4):
                    j = 4 * i + b
                    f = b % 2
                    gather_wait(j, b)

                    @pl.when(j >= 2)
                    def _():
                        scatter_wait(j - 2, f)

                    convert(b, f)
                    scatter(j, f)

                    @pl.when(j + 4 < q)
                    def _():
                        gather(j + 4, b)

                return ()

            lax.fori_loop(0, q // 4, body, ())
            scatter_wait(q - 2, 0)
            scatter_wait(q - 1, 1)

        if NC == 1:
            run(q0, si * q0)
        else:
            @pl.when(ci == 0)
            def _():
                run(q0, si * q0)

            @pl.when(ci == 1)
            def _():
                run(q1, NS * q0 + si * q1)

        plsc.subcore_barrier()

        def drain_sv(p):
            off, sz = pieces[p]
            return pltpu.async_copy(
                acc_s.at[pl.ds(si * rows_pt + off, sz)],
                fbufs[p % 2].at[pl.ds(0, sz)], gsems[p % 2])

        def drain_vh(p):
            off, sz = pieces[p]
            return pltpu.async_copy(
                fbufs[p % 2].at[pl.ds(0, sz)],
                out_hbm.at[ci, pl.ds(si * rows_pt + off, sz)], ssems[p % 2])

        P = len(pieces)
        sv = {p: drain_sv(p) for p in range(min(2, P))}
        vh = {}
        for p in range(min(2, P)):
            sv[p].wait()
            vh[p] = drain_vh(p)
        for p in range(2, P):
            vh[p - 2].wait()
            sv[p] = drain_sv(p)
            sv[p].wait()
            vh[p] = drain_vh(p)
        for p in range(max(0, P - 2), P):
            vh[p].wait()

    return k(src2, dst2, hp)


def _sc_aggregate(src2, dst2, h, n_pad, q0, q1, rows_pt):
    W = h.shape[1]
    q_max = max(q0, q1)

    @functools.partial(
        pl.kernel,
        out_type=jax.ShapeDtypeStruct((NC, n_pad, W), jnp.float32),
        mesh=_sc_mesh(),
        compiler_params=pltpu.CompilerParams(use_tc_tiling_on_sc=False),
        scratch_types=[
            pltpu.VMEM((q_max, CH), jnp.int32),
            pltpu.VMEM((q_max, CH), jnp.int32),
            [pltpu.VMEM((CH, W), jnp.float32) for _ in range(4)],
            pltpu.VMEM_SHARED((n_pad, W), jnp.float32),
            [pltpu.SemaphoreType.DMA for _ in range(4)],
            [pltpu.SemaphoreType.DMA for _ in range(4)],
        ],
    )
    def k(src_hbm, dst_hbm, h_hbm, out_hbm,
          src_v, dst_v, bufs, acc_s, gsems, ssems):
        ci = lax.axis_index("c")
        si = lax.axis_index("s")
        n_pieces = -(-rows_pt // CH)
        pieces = [(p * CH, min(CH, rows_pt - p * CH)) for p in range(n_pieces)]

        def zrow(r, _):
            for b in range(4):
                for c in range(W // 16):
                    bufs[b][r, pl.ds(c * 16, 16)] = jnp.zeros(
                        (16,), jnp.float32)
            return ()

        lax.fori_loop(0, CH, zrow, ())
        init_copies = []
        for p, (off, sz) in enumerate(pieces):
            sem = gsems[p % 4] if p < 4 else ssems[p - 4]
            cp = pltpu.async_copy(bufs[p % 4].at[pl.ds(0, sz)],
                                  acc_s.at[pl.ds(si * rows_pt + off, sz)],
                                  sem)
            init_copies.append(cp)

        def gather(j, b):
            pltpu.async_copy(h_hbm.at[src_v.at[j]], bufs[b], gsems[b])

        def gather_wait(j, b):
            pltpu.make_async_copy(
                h_hbm.at[src_v.at[j]], bufs[b], gsems[b]).wait()

        def scatter(j, b):
            pltpu.async_copy(bufs[b], acc_s.at[dst_v.at[j]], ssems[b],
                             add=True)

        def scatter_wait(j, b):
            pltpu.make_async_copy(
                bufs[b], acc_s.at[dst_v.at[j]], ssems[b]).wait()

        def run(q, start):
            pltpu.sync_copy(src_hbm.at[pl.ds(start, q)],
                            src_v.at[pl.ds(0, q)])
            pltpu.sync_copy(dst_hbm.at[pl.ds(start, q)],
                            dst_v.at[pl.ds(0, q)])
            for cp in init_copies:
                cp.wait()
            plsc.subcore_barrier()

            for b in range(4):
                gather(b, b)

            def body(i, _):
                for b in range(4):
                    j = 4 * i + b
                    gather_wait(j, b)
                    scatter(j, b)

                    @pl.when(j + 4 < q)
                    def _():
                        scatter_wait(j, b)
                        gather(j + 4, b)

                return ()

            lax.fori_loop(0, q // 4, body, ())
            for b in range(4):
                scatter_wait(q - 4 + b, b)

        if NC == 1:
            run(q0, si * q0)
        else:
            @pl.when(ci == 0)
            def _():
                run(q0, si * q0)

            @pl.when(ci == 1)
            def _():
                run(q1, NS * q0 + si * q1)

        plsc.subcore_barrier()
        def drain_sv(p):
            off, sz = pieces[p]
            return pltpu.async_copy(
                acc_s.at[pl.ds(si * rows_pt + off, sz)],
                bufs[p % 4].at[pl.ds(0, sz)], gsems[p % 4])

        def drain_vh(p):
            off, sz = pieces[p]
            return pltpu.async_copy(
                bufs[p % 4].at[pl.ds(0, sz)],
                out_hbm.at[ci, pl.ds(si * rows_pt + off, sz)], ssems[p % 4])

        P = len(pieces)
        sv = {p: drain_sv(p) for p in range(min(4, P))}
        vh = {}
        for p in range(min(4, P)):
            sv[p].wait()
            vh[p] = drain_vh(p)
        for p in range(4, P):
            vh[p - 4].wait()
            sv[p] = drain_sv(p)
            sv[p].wait()
            vh[p] = drain_vh(p)
        for p in range(max(0, P - 4), P):
            vh[p].wait()

    return k(src2, dst2, h)


def _tc_matmul1(x, W1, rb):
    n, d_in = x.shape
    d_hid = W1.shape[1]

    def body(x_ref, w_ref, h_ref):
        h_ref[...] = jnp.dot(x_ref[...], w_ref[...],
                             preferred_element_type=jnp.float32)

    return pl.pallas_call(
        body,
        grid=(n // rb,),
        in_specs=[
            pl.BlockSpec((rb, d_in), lambda i: (i, 0)),
            pl.BlockSpec((d_in, d_hid), lambda i: (0, 0)),
        ],
        out_specs=pl.BlockSpec((rb, d_hid), lambda i: (i, 0)),
        out_shape=jax.ShapeDtypeStruct((n, d_hid), jnp.float32),
    )(x, W1)


def _tc_scale1(degp, h1, rb):
    n, d_hid = h1.shape
    hw = d_hid // 2

    def body(degp_ref, h_ref, hsp_ref, dis_ref):
        deg = sum(degp_ref[c] for c in range(NC)) + 1.0
        dis = lax.rsqrt(jnp.maximum(deg, 1.0))
        hs = h_ref[...] * dis
        bits = lax.bitcast_convert_type(
            hs.astype(jnp.bfloat16), jnp.uint16).astype(jnp.uint32)
        word = jnp.bitwise_or(bits[:, :hw],
                              jnp.left_shift(bits[:, hw:], 16))
        hsp_ref[...] = lax.bitcast_convert_type(word, jnp.int32)
        dis_ref[...] = dis

    return pl.pallas_call(
        body,
        grid=(n // rb,),
        in_specs=[
            pl.BlockSpec((NC, rb, 1), lambda i: (0, i, 0)),
            pl.BlockSpec((rb, d_hid), lambda i: (i, 0)),
        ],
        out_specs=[
            pl.BlockSpec((rb, hw), lambda i: (i, 0)),
            pl.BlockSpec((rb, 1), lambda i: (i, 0)),
        ],
        out_shape=[
            jax.ShapeDtypeStruct((n, hw), jnp.int32),
            jax.ShapeDtypeStruct((n, 1), jnp.float32),
        ],
    )(degp, h1)


def _tc_layer2(aggp, hsp, dis, W2p, b1, rb, wp):
    n, hw = hsp.shape
    d_hid = 2 * hw

    def body(aggp_ref, hsp_ref, dis_ref, w_ref, b_ref, hs2_ref):
        w = hsp_ref[...]
        lo = lax.bitcast_convert_type(jnp.left_shift(w, 16), jnp.float32)
        hi = lax.bitcast_convert_type(
            jnp.bitwise_and(w, jnp.int32(-65536)), jnp.float32)
        hs = jnp.concatenate([lo, hi], axis=1)
        agg = sum(aggp_ref[c] for c in range(NC)) + hs
        r = jnp.maximum(agg * dis_ref[...] + b_ref[...], 0.0)
        h2 = jnp.dot(r, w_ref[...], preferred_element_type=jnp.float32)
        hs2_ref[...] = h2 * dis_ref[...]

    return pl.pallas_call(
        body,
        grid=(n // rb,),
        in_specs=[
            pl.BlockSpec((NC, rb, d_hid), lambda i: (0, i, 0)),
            pl.BlockSpec((rb, hw), lambda i: (i, 0)),
            pl.BlockSpec((rb, 1), lambda i: (i, 0)),
            pl.BlockSpec((d_hid, wp), lambda i: (0, 0)),
            pl.BlockSpec((1, d_hid), lambda i: (0, 0)),
        ],
        out_specs=pl.BlockSpec((rb, wp), lambda i: (i, 0)),
        out_shape=jax.ShapeDtypeStruct((n, wp), jnp.float32),
    )(aggp, hsp, dis, W2p, b1)


def _tc_final(agg2p, hs2w, dis, b2, rb, d_out):
    n, wp = hs2w.shape

    def body(ap_ref, hs2_ref, dis_ref, b_ref, out_ref):
        a = sum(ap_ref[c] for c in range(NC)) + hs2_ref[...]
        o = (a * dis_ref[...])[:, :d_out] + b_ref[...]
        m = jnp.max(o, axis=1, keepdims=True)
        lse = m + jnp.log(jnp.sum(jnp.exp(o - m), axis=1, keepdims=True))
        out_ref[...] = o - lse

    return pl.pallas_call(
        body,
        grid=(n // rb,),
        in_specs=[
            pl.BlockSpec((NC, rb, wp), lambda i: (0, i, 0)),
            pl.BlockSpec((rb, wp), lambda i: (i, 0)),
            pl.BlockSpec((rb, 1), lambda i: (i, 0)),
            pl.BlockSpec((1, d_out), lambda i: (0, 0)),
        ],
        out_specs=pl.BlockSpec((rb, d_out), lambda i: (i, 0)),
        out_shape=jax.ShapeDtypeStruct((n, d_out), jnp.float32),
    )(agg2p, hs2w, dis, b2)


def kernel(x, edge_index, W1, b1, W2, b2):
    n = x.shape[0]
    e = edge_index.shape[1]
    d_hid = W1.shape[1]
    d_out = W2.shape[1]
    wp = 16

    s_tot = -(-(-(-e // (NS * CH))) // 8) * 8
    if NC == 1:
        q0, q1 = s_tot, 0
    else:
        q0 = max(4, int(round(s_tot * SPLIT0 / 4)) * 4)
        q1 = s_tot - q0
    e_pad = NS * s_tot * CH
    n_pad = -(-(n + 1) // (NS * 16)) * (NS * 16)
    rows_pt = n_pad // NS
    rb = 1000 if n % 1000 == 0 else 8

    src = edge_index[0]
    dst = edge_index[1]
    src2 = jnp.concatenate(
        [src, jnp.zeros((e_pad - e,), jnp.int32)]).reshape(NS * s_tot, CH)
    dst2 = jnp.concatenate(
        [dst, jnp.full((e_pad - e,), n, jnp.int32)]).reshape(NS * s_tot, CH)

    W2p = jnp.pad(W2, ((0, 0), (0, wp - d_out)))

    h1 = _tc_matmul1(x, W1, rb)
    degp = _sc_degree(dst2, n_pad, q0, q1, rows_pt)
    degp3 = degp[:, :n, None]
    hsp, dis = _tc_scale1(degp3, h1, rb)
    aggp = _sc_aggregate_packed(src2, dst2, hsp, n_pad, q0, q1, rows_pt)
    hs2w = _tc_layer2(aggp, hsp, dis, W2p, b1.reshape(1, d_hid), rb, wp)
    agg2p = _sc_aggregate(src2, dst2, hs2w, n_pad, q0, q1, rows_pt)
    return _tc_final(agg2p, hs2w, dis, b2.reshape(1, d_out), rb, d_out)
